# baseline (speedup 1.0000x reference)
import functools

import jax
import jax.numpy as jnp
from jax import lax
from jax.experimental import pallas as pl
from jax.experimental.pallas import tpu as pltpu
from jax.experimental.pallas import tpu_sc as plsc

N = 10000
E = 320000
D = 128
H1 = 32
H2 = 16

NC = 2
NS = 16
NW = NC * NS
K = 128
CH = E // K
C1 = -(-CH // NW)
C2 = CH - (NW - 1) * C1
NBUF = 4
RPT = 632
NP = NS * RPT
NA = NP

_MESH = dict(core_axis_name="c", subcore_axis_name="s")
_SC_PARAMS = pltpu.CompilerParams(use_tc_tiling_on_sc=False)
_MAGIC = 0x5F3759DF


def _rsqrt16(dv):
  ii = lax.bitcast_convert_type(dv, jnp.int32)
  ii = _MAGIC - lax.shift_right_logical(ii, 1)
  g = lax.bitcast_convert_type(ii, jnp.float32)
  g = g * (1.5 - 0.5 * dv * g * g)
  g = g * (1.5 - 0.5 * dv * g * g)
  g = g * (1.5 - 0.5 * dv * g * g)
  return g


@functools.partial(
    pl.kernel,
    out_type=[
        jax.ShapeDtypeStruct((NC, NP, H1), jnp.float32),
        jax.ShapeDtypeStruct((NP, 16), jnp.float32),
    ],
    mesh=plsc.VectorSubcoreMesh(**_MESH),
    scratch_types=[
        pltpu.VMEM((C1, K), jnp.int32),
        pltpu.VMEM((C1, K), jnp.int32),
        pltpu.VMEM((NBUF, K, H1), jnp.float32),
        pltpu.VMEM((RPT, H1), jnp.float32),
        pltpu.VMEM((RPT, 16), jnp.float32),
        pltpu.VMEM((RPT, 16), jnp.float32),
        pltpu.VMEM_SHARED((NA, H1), jnp.float32),
        pltpu.VMEM_SHARED((NA, H1), jnp.float32),
        pltpu.SemaphoreType.DMA,
        pltpu.SemaphoreType.DMA,
    ],
    compiler_params=_SC_PARAMS,
)
def _agg32(u, degp, ei3, zeros_h, out, dis_out,
           sidx, didx, rows, uloc, d0, d1, acc, ysp, gsem, ssem):
  cid = lax.axis_index("c")
  sid = lax.axis_index("s")
  wid = sid * NC + cid
  cbase = wid * C1
  is_last = wid == NW - 1
  n = jnp.where(is_last, C2, C1)
  slab = pl.ds(sid * RPT, RPT)

  @pl.when(jnp.logical_not(is_last))
  def _():
    pltpu.sync_copy(ei3.at[0, pl.ds(cbase, C1)], sidx)
    pltpu.sync_copy(ei3.at[1, pl.ds(cbase, C1)], didx)

  @pl.when(is_last)
  def _():
    pltpu.sync_copy(ei3.at[0, pl.ds(cbase, C2)], sidx.at[pl.ds(0, C2)])
    pltpu.sync_copy(ei3.at[1, pl.ds(cbase, C2)], didx.at[pl.ds(0, C2)])

  pltpu.sync_copy(u.at[slab], uloc)
  pltpu.sync_copy(degp.at[0, slab], d0)
  pltpu.sync_copy(degp.at[1, slab], d1)

  def prow(r, carry):
    g = _rsqrt16(d0[r] + d1[r] + 1.0)
    d0[r] = g
    for t in range(H1 // 16):
      uloc[r, pl.ds(16 * t, 16)] = uloc[r, pl.ds(16 * t, 16)] * g
    return carry

  lax.fori_loop(0, RPT, prow, 0)

  pltpu.sync_copy(uloc, ysp.at[slab])

  @pl.when(cid == 0)
  def _():
    pltpu.sync_copy(uloc, acc.at[slab])
    pltpu.sync_copy(d0, dis_out.at[slab])

  @pl.when(cid != 0)
  def _():
    pltpu.sync_copy(zeros_h.at[slab], acc.at[slab])

  plsc.subcore_barrier()

  pltpu.async_copy(ysp.at[sidx.at[0]], rows.at[0], gsem)
  pltpu.async_copy(ysp.at[sidx.at[1]], rows.at[1], gsem)

  def body(j, carry):
    b = lax.rem(j, NBUF)

    @pl.when(j >= 2)
    def _():
      pltpu.make_async_copy(
          rows.at[lax.rem(j + 2, NBUF)],
          acc.at[didx.at[j - 2]], ssem).wait()

    @pl.when(j + 2 < n)
    def _():
      pltpu.async_copy(
          ysp.at[sidx.at[j + 2]], rows.at[lax.rem(j + 2, NBUF)], gsem)

    pltpu.make_async_copy(ysp.at[sidx.at[j]], rows.at[b], gsem).wait()
    pltpu.async_copy(rows.at[b], acc.at[didx.at[j]], ssem, add=True)
    return carry

  lax.fori_loop(0, n, body, 0)
  pltpu.make_async_copy(rows.at[0], acc.at[didx.at[0]], ssem).wait()
  pltpu.make_async_copy(rows.at[0], acc.at[didx.at[0]], ssem).wait()
  plsc.subcore_barrier()
  pltpu.sync_copy(acc.at[slab], out.at[cid, slab])


@functools.partial(
    pl.kernel,
    out_type=jax.ShapeDtypeStruct((NC, NP, H2), jnp.float32),
    mesh=plsc.VectorSubcoreMesh(**_MESH),
    scratch_types=[
        pltpu.VMEM((C1, K), jnp.int32),
        pltpu.VMEM((C1, K), jnp.int32),
        pltpu.VMEM((NBUF, K, H2), jnp.float32),
        pltpu.VMEM((RPT, H2), jnp.float32),
        pltpu.VMEM((RPT, 16), jnp.float32),
        pltpu.VMEM_SHARED((NA, H2), jnp.float32),
        pltpu.VMEM_SHARED((NA, H2), jnp.float32),
        pltpu.SemaphoreType.DMA,
        pltpu.SemaphoreType.DMA,
    ],
    compiler_params=_SC_PARAMS,
)
def _agg16(u, dis16, ei3, zeros_h, out,
           sidx, didx, rows, uloc, dloc, acc, ysp, gsem, ssem):
  cid = lax.axis_index("c")
  sid = lax.axis_index("s")
  wid = sid * NC + cid
  cbase = wid * C1
  is_last = wid == NW - 1
  n = jnp.where(is_last, C2, C1)
  slab = pl.ds(sid * RPT, RPT)

  @pl.when(jnp.logical_not(is_last))
  def _():
    pltpu.sync_copy(ei3.at[0, pl.ds(cbase, C1)], sidx)
    pltpu.sync_copy(ei3.at[1, pl.ds(cbase, C1)], didx)

  @pl.when(is_last)
  def _():
    pltpu.sync_copy(ei3.at[0, pl.ds(cbase, C2)], sidx.at[pl.ds(0, C2)])
    pltpu.sync_copy(ei3.at[1, pl.ds(cbase, C2)], didx.at[pl.ds(0, C2)])

  pltpu.sync_copy(u.at[slab], uloc)
  pltpu.sync_copy(dis16.at[slab], dloc)

  def prow(r, carry):
    uloc[r] = uloc[r] * dloc[r]
    return carry

  lax.fori_loop(0, RPT, prow, 0)

  pltpu.sync_copy(uloc, ysp.at[slab])

  @pl.when(cid == 0)
  def _():
    pltpu.sync_copy(uloc, acc.at[slab])

  @pl.when(cid != 0)
  def _():
    pltpu.sync_copy(zeros_h.at[slab], acc.at[slab])

  plsc.subcore_barrier()

  pltpu.async_copy(ysp.at[sidx.at[0]], rows.at[0], gsem)
  pltpu.async_copy(ysp.at[sidx.at[1]], rows.at[1], gsem)

  def body(j, carry):
    b = lax.rem(j, NBUF)

    @pl.when(j >= 2)
    def _():
      pltpu.make_async_copy(
          rows.at[lax.rem(j + 2, NBUF)],
          acc.at[didx.at[j - 2]], ssem).wait()

    @pl.when(j + 2 < n)
    def _():
      pltpu.async_copy(
          ysp.at[sidx.at[j + 2]], rows.at[lax.rem(j + 2, NBUF)], gsem)

    pltpu.make_async_copy(ysp.at[sidx.at[j]], rows.at[b], gsem).wait()
    pltpu.async_copy(rows.at[b], acc.at[didx.at[j]], ssem, add=True)
    return carry

  lax.fori_loop(0, n, body, 0)
  pltpu.make_async_copy(rows.at[0], acc.at[didx.at[0]], ssem).wait()
  pltpu.make_async_copy(rows.at[0], acc.at[didx.at[0]], ssem).wait()
  plsc.subcore_barrier()
  pltpu.sync_copy(acc.at[slab], out.at[cid, slab])


@functools.partial(
    pl.kernel,
    out_type=jax.ShapeDtypeStruct((NC, NP, 16), jnp.float32),
    mesh=plsc.VectorSubcoreMesh(**_MESH),
    scratch_types=[
        pltpu.VMEM((C1, K), jnp.int32),
        pltpu.VMEM((K, 16), jnp.float32),
        pltpu.VMEM_SHARED((NA, 16), jnp.float32),
        pltpu.SemaphoreType.DMA,
    ],
    compiler_params=_SC_PARAMS,
)
def _deg_sc(zeros_nh, ones_kh, ei3, out, didx, ones_v, acc, ssem):
  cid = lax.axis_index("c")
  sid = lax.axis_index("s")
  wid = sid * NC + cid
  cbase = wid * C1
  is_last = wid == NW - 1
  n = jnp.where(is_last, C2, C1)

  @pl.when(jnp.logical_not(is_last))
  def _():
    pltpu.sync_copy(ei3.at[1, pl.ds(cbase, C1)], didx)

  @pl.when(is_last)
  def _():
    pltpu.sync_copy(ei3.at[1, pl.ds(cbase, C2)], didx.at[pl.ds(0, C2)])

  pltpu.sync_copy(ones_kh, ones_v)
  pltpu.sync_copy(zeros_nh.at[pl.ds(sid * RPT, RPT)],
                  acc.at[pl.ds(sid * RPT, RPT)])
  plsc.subcore_barrier()

  def body(j, carry):
    @pl.when(j >= 4)
    def _():
      pltpu.make_async_copy(ones_v, acc.at[didx.at[0]], ssem).wait()

    pltpu.async_copy(ones_v, acc.at[didx.at[j]], ssem, add=True)
    return carry

  lax.fori_loop(0, n, body, 0)
  for _ in range(4):
    pltpu.make_async_copy(ones_v, acc.at[didx.at[0]], ssem).wait()
  plsc.subcore_barrier()
  pltpu.sync_copy(acc.at[pl.ds(sid * RPT, RPT)],
                  out.at[cid, pl.ds(sid * RPT, RPT)])


def _tc0_body(x, w1, u_ref):
  u_ref[0:N, :] = jnp.dot(x[...], w1[...],
                          preferred_element_type=jnp.float32)


def _tc1_body(degp, dis_ref):
  deg = degp[0, :N, 0:1] + degp[1, :N, 0:1] + 1.0
  dis_ref[...] = 1.0 / jnp.sqrt(deg)


def _tc2_body(zp, dis, b1, w2, u2_ref):
  z = zp[0, :N] + zp[1, :N]
  h = jnp.maximum(z * dis[...] + b1[...], 0.0)
  u2_ref[0:N, :] = jnp.dot(h, w2[...], preferred_element_type=jnp.float32)


def _tc3_body(zp, dis, b2, wout, bout, emb_ref, logit_ref):
  z = zp[0, :N] + zp[1, :N]
  emb = jnp.maximum(z * dis[...] + b2[...], 0.0)
  emb_ref[...] = emb
  logit_ref[...] = jnp.dot(emb, wout[...],
                           preferred_element_type=jnp.float32) + bout[0, 0]


def kernel(x, edge_index, W1, b1, W2, b2, Wout, bout):
  ei3 = edge_index.reshape(2, CH, K)
  zeros_16 = jnp.zeros((NP, 16), jnp.float32)
  zeros_32 = jnp.zeros((NP, H1), jnp.float32)
  ones_kh = jnp.ones((K, 16), jnp.float32)

  degp = _deg_sc(zeros_16, ones_kh, ei3)

  u1 = pl.pallas_call(
      _tc0_body,
      out_shape=jax.ShapeDtypeStruct((NP, H1), jnp.float32),
  )(x, W1)

  zp1, dis16 = _agg32(u1, degp, ei3, zeros_32)

  dis = pl.pallas_call(
      _tc1_body,
      out_shape=jax.ShapeDtypeStruct((N, 1), jnp.float32),
  )(degp)

  u2 = pl.pallas_call(
      _tc2_body,
      out_shape=jax.ShapeDtypeStruct((NP, H2), jnp.float32),
  )(zp1, dis, b1.reshape(1, H1), W2)

  zp2 = _agg16(u2, dis16, ei3, zeros_16)

  embedding, logits = pl.pallas_call(
      _tc3_body,
      out_shape=[
          jax.ShapeDtypeStruct((N, H2), jnp.float32),
          jax.ShapeDtypeStruct((N, 1), jnp.float32),
      ],
  )(zp2, dis, b2.reshape(1, H2), Wout, bout.reshape(1, 1))

  return (logits.squeeze(-1), embedding)


# speedup vs baseline: 1.0310x; 1.0310x over previous
import functools

import jax
import jax.numpy as jnp
from jax import lax
from jax.experimental import pallas as pl
from jax.experimental.pallas import tpu as pltpu
from jax.experimental.pallas import tpu_sc as plsc

N = 10000
E = 320000
D = 128
H1 = 32
H2 = 16

NC = 2
NS = 16
NW = NC * NS
K = 128
CH = E // K
C1 = -(-CH // NW)
C2 = CH - (NW - 1) * C1
NBUF = 4
RPT = 632
NP = NS * RPT
NA = NP

_MESH = dict(core_axis_name="c", subcore_axis_name="s")
_SC_PARAMS = pltpu.CompilerParams(use_tc_tiling_on_sc=False)
_MAGIC = 0x5F3759DF


def _rsqrt16(dv):
  ii = lax.bitcast_convert_type(dv, jnp.int32)
  ii = _MAGIC - lax.shift_right_logical(ii, 1)
  g = lax.bitcast_convert_type(ii, jnp.float32)
  g = g * (1.5 - 0.5 * dv * g * g)
  g = g * (1.5 - 0.5 * dv * g * g)
  g = g * (1.5 - 0.5 * dv * g * g)
  return g


@functools.partial(
    pl.kernel,
    out_type=[
        jax.ShapeDtypeStruct((NC, NP, H1), jnp.float32),
        jax.ShapeDtypeStruct((NP, 16), jnp.float32),
    ],
    mesh=plsc.VectorSubcoreMesh(**_MESH),
    scratch_types=[
        pltpu.VMEM((C1 * K,), jnp.int32),
        pltpu.VMEM((C1 * K,), jnp.int32),
        pltpu.VMEM((NBUF, K, H1), jnp.float32),
        pltpu.VMEM((RPT, H1), jnp.float32),
        pltpu.VMEM((RPT, 16), jnp.float32),
        pltpu.VMEM((RPT, 16), jnp.float32),
        pltpu.VMEM_SHARED((NA, H1), jnp.float32),
        pltpu.VMEM_SHARED((NA, H1), jnp.float32),
        pltpu.SemaphoreType.DMA,
        pltpu.SemaphoreType.DMA,
        pltpu.SemaphoreType.DMA,
    ],
    compiler_params=_SC_PARAMS,
)
def _agg32(u, degp, ei, zeros_h, out, dis_out,
           sidx, didx, rows, uloc, d0, d1, acc, ysp, gsem, ssem, isem):
  cid = lax.axis_index("c")
  sid = lax.axis_index("s")
  wid = sid * NC + cid
  ebase = wid * C1 * K
  is_last = wid == NW - 1
  n = jnp.where(is_last, C2, C1)
  slab = pl.ds(sid * RPT, RPT)

  @pl.when(jnp.logical_not(is_last))
  def _():
    pltpu.async_copy(ei.at[0, pl.ds(ebase, C1 * K)], sidx, isem)
    pltpu.async_copy(ei.at[1, pl.ds(ebase, C1 * K)], didx, isem)

  @pl.when(is_last)
  def _():
    pltpu.async_copy(ei.at[0, pl.ds(ebase, C2 * K)],
                     sidx.at[pl.ds(0, C2 * K)], isem)
    pltpu.async_copy(ei.at[1, pl.ds(ebase, C2 * K)],
                     didx.at[pl.ds(0, C2 * K)], isem)

  pltpu.sync_copy(u.at[slab], uloc)
  pltpu.sync_copy(degp.at[0, slab], d0)
  pltpu.sync_copy(degp.at[1, slab], d1)

  def prow(r, carry):
    g = _rsqrt16(d0[r] + d1[r] + 1.0)
    d0[r] = g
    for t in range(H1 // 16):
      uloc[r, pl.ds(16 * t, 16)] = uloc[r, pl.ds(16 * t, 16)] * g
    return carry

  lax.fori_loop(0, RPT, prow, 0)

  pltpu.sync_copy(uloc, ysp.at[slab])

  @pl.when(cid == 0)
  def _():
    pltpu.sync_copy(uloc, acc.at[slab])
    pltpu.sync_copy(d0, dis_out.at[slab])

  @pl.when(cid != 0)
  def _():
    pltpu.sync_copy(zeros_h.at[slab], acc.at[slab])

  @pl.when(jnp.logical_not(is_last))
  def _():
    pltpu.make_async_copy(ei.at[0, pl.ds(ebase, C1 * K)], sidx, isem).wait()
    pltpu.make_async_copy(ei.at[1, pl.ds(ebase, C1 * K)], didx, isem).wait()

  @pl.when(is_last)
  def _():
    pltpu.make_async_copy(ei.at[0, pl.ds(ebase, C2 * K)],
                          sidx.at[pl.ds(0, C2 * K)], isem).wait()
    pltpu.make_async_copy(ei.at[1, pl.ds(ebase, C2 * K)],
                          didx.at[pl.ds(0, C2 * K)], isem).wait()

  plsc.subcore_barrier()

  pltpu.async_copy(ysp.at[sidx.at[pl.ds(0, K)]], rows.at[0], gsem)
  pltpu.async_copy(ysp.at[sidx.at[pl.ds(K, K)]], rows.at[1], gsem)

  def body(j, carry):
    b = lax.rem(j, NBUF)

    @pl.when(j >= 2)
    def _():
      pltpu.make_async_copy(
          rows.at[lax.rem(j + 2, NBUF)],
          acc.at[didx.at[pl.ds((j - 2) * K, K)]], ssem).wait()

    @pl.when(j + 2 < n)
    def _():
      pltpu.async_copy(
          ysp.at[sidx.at[pl.ds((j + 2) * K, K)]],
          rows.at[lax.rem(j + 2, NBUF)], gsem)

    pltpu.make_async_copy(
        ysp.at[sidx.at[pl.ds(j * K, K)]], rows.at[b], gsem).wait()
    pltpu.async_copy(rows.at[b], acc.at[didx.at[pl.ds(j * K, K)]], ssem,
                     add=True)
    return carry

  lax.fori_loop(0, n, body, 0)
  pltpu.make_async_copy(
      rows.at[0], acc.at[didx.at[pl.ds(0, K)]], ssem).wait()
  pltpu.make_async_copy(
      rows.at[0], acc.at[didx.at[pl.ds(0, K)]], ssem).wait()
  plsc.subcore_barrier()
  pltpu.sync_copy(acc.at[slab], out.at[cid, slab])


@functools.partial(
    pl.kernel,
    out_type=jax.ShapeDtypeStruct((NC, NP, H2), jnp.float32),
    mesh=plsc.VectorSubcoreMesh(**_MESH),
    scratch_types=[
        pltpu.VMEM((C1 * K,), jnp.int32),
        pltpu.VMEM((C1 * K,), jnp.int32),
        pltpu.VMEM((NBUF, K, H2), jnp.float32),
        pltpu.VMEM((RPT, H2), jnp.float32),
        pltpu.VMEM((RPT, 16), jnp.float32),
        pltpu.VMEM_SHARED((NA, H2), jnp.float32),
        pltpu.VMEM_SHARED((NA, H2), jnp.float32),
        pltpu.SemaphoreType.DMA,
        pltpu.SemaphoreType.DMA,
        pltpu.SemaphoreType.DMA,
    ],
    compiler_params=_SC_PARAMS,
)
def _agg16(u, dis16, ei, zeros_h, out,
           sidx, didx, rows, uloc, dloc, acc, ysp, gsem, ssem, isem):
  cid = lax.axis_index("c")
  sid = lax.axis_index("s")
  wid = sid * NC + cid
  ebase = wid * C1 * K
  is_last = wid == NW - 1
  n = jnp.where(is_last, C2, C1)
  slab = pl.ds(sid * RPT, RPT)

  @pl.when(jnp.logical_not(is_last))
  def _():
    pltpu.async_copy(ei.at[0, pl.ds(ebase, C1 * K)], sidx, isem)
    pltpu.async_copy(ei.at[1, pl.ds(ebase, C1 * K)], didx, isem)

  @pl.when(is_last)
  def _():
    pltpu.async_copy(ei.at[0, pl.ds(ebase, C2 * K)],
                     sidx.at[pl.ds(0, C2 * K)], isem)
    pltpu.async_copy(ei.at[1, pl.ds(ebase, C2 * K)],
                     didx.at[pl.ds(0, C2 * K)], isem)

  pltpu.sync_copy(u.at[slab], uloc)
  pltpu.sync_copy(dis16.at[slab], dloc)

  def prow(r, carry):
    uloc[r] = uloc[r] * dloc[r]
    return carry

  lax.fori_loop(0, RPT, prow, 0)

  pltpu.sync_copy(uloc, ysp.at[slab])

  @pl.when(cid == 0)
  def _():
    pltpu.sync_copy(uloc, acc.at[slab])

  @pl.when(cid != 0)
  def _():
    pltpu.sync_copy(zeros_h.at[slab], acc.at[slab])

  @pl.when(jnp.logical_not(is_last))
  def _():
    pltpu.make_async_copy(ei.at[0, pl.ds(ebase, C1 * K)], sidx, isem).wait()
    pltpu.make_async_copy(ei.at[1, pl.ds(ebase, C1 * K)], didx, isem).wait()

  @pl.when(is_last)
  def _():
    pltpu.make_async_copy(ei.at[0, pl.ds(ebase, C2 * K)],
                          sidx.at[pl.ds(0, C2 * K)], isem).wait()
    pltpu.make_async_copy(ei.at[1, pl.ds(ebase, C2 * K)],
                          didx.at[pl.ds(0, C2 * K)], isem).wait()

  plsc.subcore_barrier()

  pltpu.async_copy(ysp.at[sidx.at[pl.ds(0, K)]], rows.at[0], gsem)
  pltpu.async_copy(ysp.at[sidx.at[pl.ds(K, K)]], rows.at[1], gsem)

  def body(j, carry):
    b = lax.rem(j, NBUF)

    @pl.when(j >= 2)
    def _():
      pltpu.make_async_copy(
          rows.at[lax.rem(j + 2, NBUF)],
          acc.at[didx.at[pl.ds((j - 2) * K, K)]], ssem).wait()

    @pl.when(j + 2 < n)
    def _():
      pltpu.async_copy(
          ysp.at[sidx.at[pl.ds((j + 2) * K, K)]],
          rows.at[lax.rem(j + 2, NBUF)], gsem)

    pltpu.make_async_copy(
        ysp.at[sidx.at[pl.ds(j * K, K)]], rows.at[b], gsem).wait()
    pltpu.async_copy(rows.at[b], acc.at[didx.at[pl.ds(j * K, K)]], ssem,
                     add=True)
    return carry

  lax.fori_loop(0, n, body, 0)
  pltpu.make_async_copy(
      rows.at[0], acc.at[didx.at[pl.ds(0, K)]], ssem).wait()
  pltpu.make_async_copy(
      rows.at[0], acc.at[didx.at[pl.ds(0, K)]], ssem).wait()
  plsc.subcore_barrier()
  pltpu.sync_copy(acc.at[slab], out.at[cid, slab])


@functools.partial(
    pl.kernel,
    out_type=jax.ShapeDtypeStruct((NC, NP, 16), jnp.float32),
    mesh=plsc.VectorSubcoreMesh(**_MESH),
    scratch_types=[
        pltpu.VMEM((C1 * K,), jnp.int32),
        pltpu.VMEM((K, 16), jnp.float32),
        pltpu.VMEM_SHARED((NA, 16), jnp.float32),
        pltpu.SemaphoreType.DMA,
        pltpu.SemaphoreType.DMA,
    ],
    compiler_params=_SC_PARAMS,
)
def _deg_sc(zeros_nh, ones_kh, ei, out, didx, ones_v, acc, ssem, isem):
  cid = lax.axis_index("c")
  sid = lax.axis_index("s")
  wid = sid * NC + cid
  ebase = wid * C1 * K
  is_last = wid == NW - 1
  n = jnp.where(is_last, C2, C1)

  @pl.when(jnp.logical_not(is_last))
  def _():
    pltpu.async_copy(ei.at[1, pl.ds(ebase, C1 * K)], didx, isem)

  @pl.when(is_last)
  def _():
    pltpu.async_copy(ei.at[1, pl.ds(ebase, C2 * K)],
                     didx.at[pl.ds(0, C2 * K)], isem)

  pltpu.sync_copy(ones_kh, ones_v)
  pltpu.sync_copy(zeros_nh.at[pl.ds(sid * RPT, RPT)],
                  acc.at[pl.ds(sid * RPT, RPT)])

  @pl.when(jnp.logical_not(is_last))
  def _():
    pltpu.make_async_copy(ei.at[1, pl.ds(ebase, C1 * K)], didx, isem).wait()

  @pl.when(is_last)
  def _():
    pltpu.make_async_copy(ei.at[1, pl.ds(ebase, C2 * K)],
                          didx.at[pl.ds(0, C2 * K)], isem).wait()

  plsc.subcore_barrier()

  def body(j, carry):
    @pl.when(j >= 4)
    def _():
      pltpu.make_async_copy(
          ones_v, acc.at[didx.at[pl.ds(0, K)]], ssem).wait()

    pltpu.async_copy(ones_v, acc.at[didx.at[pl.ds(j * K, K)]], ssem,
                     add=True)
    return carry

  lax.fori_loop(0, n, body, 0)
  for _ in range(4):
    pltpu.make_async_copy(ones_v, acc.at[didx.at[pl.ds(0, K)]], ssem).wait()
  plsc.subcore_barrier()
  pltpu.sync_copy(acc.at[pl.ds(sid * RPT, RPT)],
                  out.at[cid, pl.ds(sid * RPT, RPT)])


def _tc0_body(x, w1, u_ref):
  u_ref[0:N, :] = jnp.dot(x[...], w1[...],
                          preferred_element_type=jnp.float32)


def _tc1_body(degp, dis_ref):
  deg = degp[0, :N, 0:1] + degp[1, :N, 0:1] + 1.0
  dis_ref[...] = 1.0 / jnp.sqrt(deg)


def _tc2_body(zp, dis, b1, w2, u2_ref):
  z = zp[0, :N] + zp[1, :N]
  h = jnp.maximum(z * dis[...] + b1[...], 0.0)
  u2_ref[0:N, :] = jnp.dot(h, w2[...], preferred_element_type=jnp.float32)


def _tc3_body(zp, dis, b2, wout, bout, emb_ref, logit_ref):
  z = zp[0, :N] + zp[1, :N]
  emb = jnp.maximum(z * dis[...] + b2[...], 0.0)
  emb_ref[...] = emb
  logit_ref[...] = jnp.dot(emb, wout[...],
                           preferred_element_type=jnp.float32) + bout[0, 0]


def kernel(x, edge_index, W1, b1, W2, b2, Wout, bout):
  zeros_16 = jnp.zeros((NP, 16), jnp.float32)
  zeros_32 = jnp.zeros((NP, H1), jnp.float32)
  ones_kh = jnp.ones((K, 16), jnp.float32)

  degp = _deg_sc(zeros_16, ones_kh, edge_index)

  u1 = pl.pallas_call(
      _tc0_body,
      out_shape=jax.ShapeDtypeStruct((NP, H1), jnp.float32),
  )(x, W1)

  zp1, dis16 = _agg32(u1, degp, edge_index, zeros_32)

  dis = pl.pallas_call(
      _tc1_body,
      out_shape=jax.ShapeDtypeStruct((N, 1), jnp.float32),
  )(degp)

  u2 = pl.pallas_call(
      _tc2_body,
      out_shape=jax.ShapeDtypeStruct((NP, H2), jnp.float32),
  )(zp1, dis, b1.reshape(1, H1), W2)

  zp2 = _agg16(u2, dis16, edge_index, zeros_16)

  embedding, logits = pl.pallas_call(
      _tc3_body,
      out_shape=[
          jax.ShapeDtypeStruct((N, H2), jnp.float32),
          jax.ShapeDtypeStruct((N, 1), jnp.float32),
      ],
  )(zp2, dis, b2.reshape(1, H2), Wout, bout.reshape(1, 1))

  return (logits.squeeze(-1), embedding)


# speedup vs baseline: 1.2048x; 1.1685x over previous
import functools

import jax
import jax.numpy as jnp
from jax import lax
from jax.experimental import pallas as pl
from jax.experimental.pallas import tpu as pltpu
from jax.experimental.pallas import tpu_sc as plsc

N = 10000
E = 320000
D = 128
H1 = 32
H2 = 16

NC = 2
NS = 16
NW = NC * NS
K = 128
CH = E // K
C1 = -(-CH // NW)
C2 = CH - (NW - 1) * C1
NBUF = 4
RPT = 632
NP = NS * RPT
NA = NP

_MESH = dict(core_axis_name="c", subcore_axis_name="s")
_SC_PARAMS = pltpu.CompilerParams(use_tc_tiling_on_sc=False)
_MAGIC = 0x5F3759DF


def _rsqrt16(dv):
  ii = lax.bitcast_convert_type(dv, jnp.int32)
  ii = _MAGIC - lax.shift_right_logical(ii, 1)
  g = lax.bitcast_convert_type(ii, jnp.float32)
  g = g * (1.5 - 0.5 * dv * g * g)
  g = g * (1.5 - 0.5 * dv * g * g)
  g = g * (1.5 - 0.5 * dv * g * g)
  return g


@functools.partial(
    pl.kernel,
    out_type=[
        jax.ShapeDtypeStruct((NC, NP, H1), jnp.float32),
        jax.ShapeDtypeStruct((NP, 16), jnp.float32),
    ],
    mesh=plsc.VectorSubcoreMesh(**_MESH),
    scratch_types=[
        pltpu.VMEM((C1 * K,), jnp.int32),
        pltpu.VMEM((C1 * K,), jnp.int32),
        pltpu.VMEM((NBUF, K, H1), jnp.float32),
        pltpu.VMEM((RPT, H1), jnp.float32),
        pltpu.VMEM((RPT, 16), jnp.float32),
        pltpu.VMEM((RPT, 16), jnp.float32),
        pltpu.VMEM_SHARED((NA, H1), jnp.float32),
        pltpu.VMEM_SHARED((NA, H1), jnp.float32),
        pltpu.SemaphoreType.DMA,
        pltpu.SemaphoreType.DMA,
        pltpu.SemaphoreType.DMA,
    ],
    compiler_params=_SC_PARAMS,
)
def _agg32(u, degp, ei, zeros_h, out, dis_out,
           sidx, didx, rows, uloc, d0, d1, acc, ysp, gsem, ssem, isem):
  cid = lax.axis_index("c")
  sid = lax.axis_index("s")
  wid = sid * NC + cid
  ebase = wid * C1 * K
  is_last = wid == NW - 1
  n = jnp.where(is_last, C2, C1)
  slab = pl.ds(sid * RPT, RPT)

  @pl.when(jnp.logical_not(is_last))
  def _():
    pltpu.async_copy(ei.at[0, pl.ds(ebase, C1 * K)], sidx, isem)
    pltpu.async_copy(ei.at[1, pl.ds(ebase, C1 * K)], didx, isem)

  @pl.when(is_last)
  def _():
    pltpu.async_copy(ei.at[0, pl.ds(ebase, C2 * K)],
                     sidx.at[pl.ds(0, C2 * K)], isem)
    pltpu.async_copy(ei.at[1, pl.ds(ebase, C2 * K)],
                     didx.at[pl.ds(0, C2 * K)], isem)

  pltpu.sync_copy(u.at[slab], uloc)
  pltpu.sync_copy(degp.at[0, slab], d0)
  pltpu.sync_copy(degp.at[1, slab], d1)

  def prow(r, carry):
    g = _rsqrt16(d0[r] + d1[r] + 1.0)
    d0[r] = g
    for t in range(H1 // 16):
      uloc[r, pl.ds(16 * t, 16)] = uloc[r, pl.ds(16 * t, 16)] * g
    return carry

  lax.fori_loop(0, RPT, prow, 0)

  pltpu.sync_copy(uloc, ysp.at[slab])

  @pl.when(cid == 0)
  def _():
    pltpu.sync_copy(uloc, acc.at[slab])
    pltpu.sync_copy(d0, dis_out.at[slab])

  @pl.when(cid != 0)
  def _():
    pltpu.sync_copy(zeros_h.at[slab], acc.at[slab])

  @pl.when(jnp.logical_not(is_last))
  def _():
    pltpu.make_async_copy(ei.at[0, pl.ds(ebase, C1 * K)], sidx, isem).wait()
    pltpu.make_async_copy(ei.at[1, pl.ds(ebase, C1 * K)], didx, isem).wait()

  @pl.when(is_last)
  def _():
    pltpu.make_async_copy(ei.at[0, pl.ds(ebase, C2 * K)],
                          sidx.at[pl.ds(0, C2 * K)], isem).wait()
    pltpu.make_async_copy(ei.at[1, pl.ds(ebase, C2 * K)],
                          didx.at[pl.ds(0, C2 * K)], isem).wait()

  plsc.subcore_barrier()

  pltpu.async_copy(ysp.at[sidx.at[pl.ds(0, K)]], rows.at[0], gsem)
  pltpu.async_copy(ysp.at[sidx.at[pl.ds(K, K)]], rows.at[1], gsem)

  def body(j, carry):
    b = lax.rem(j, NBUF)

    @pl.when(j >= 2)
    def _():
      pltpu.make_async_copy(
          rows.at[lax.rem(j + 2, NBUF)],
          acc.at[didx.at[pl.ds((j - 2) * K, K)]], ssem).wait()

    @pl.when(j + 2 < n)
    def _():
      pltpu.async_copy(
          ysp.at[sidx.at[pl.ds((j + 2) * K, K)]],
          rows.at[lax.rem(j + 2, NBUF)], gsem)

    pltpu.make_async_copy(
        ysp.at[sidx.at[pl.ds(j * K, K)]], rows.at[b], gsem).wait()
    pltpu.async_copy(rows.at[b], acc.at[didx.at[pl.ds(j * K, K)]], ssem,
                     add=True)
    return carry

  lax.fori_loop(0, n, body, 0)
  pltpu.make_async_copy(
      rows.at[0], acc.at[didx.at[pl.ds(0, K)]], ssem).wait()
  pltpu.make_async_copy(
      rows.at[0], acc.at[didx.at[pl.ds(0, K)]], ssem).wait()
  plsc.subcore_barrier()
  pltpu.sync_copy(acc.at[slab], out.at[cid, slab])


@functools.partial(
    pl.kernel,
    out_type=jax.ShapeDtypeStruct((NC, NP, H2), jnp.float32),
    mesh=plsc.VectorSubcoreMesh(**_MESH),
    scratch_types=[
        pltpu.VMEM((C1 * K,), jnp.int32),
        pltpu.VMEM((C1 * K,), jnp.int32),
        pltpu.VMEM((NBUF, K, H2), jnp.float32),
        pltpu.VMEM((RPT, H2), jnp.float32),
        pltpu.VMEM((RPT, 16), jnp.float32),
        pltpu.VMEM_SHARED((NA, H2), jnp.float32),
        pltpu.VMEM_SHARED((NA, H2), jnp.float32),
        pltpu.SemaphoreType.DMA,
        pltpu.SemaphoreType.DMA,
        pltpu.SemaphoreType.DMA,
    ],
    compiler_params=_SC_PARAMS,
)
def _agg16(u, dis16, ei, zeros_h, out,
           sidx, didx, rows, uloc, dloc, acc, ysp, gsem, ssem, isem):
  cid = lax.axis_index("c")
  sid = lax.axis_index("s")
  wid = sid * NC + cid
  ebase = wid * C1 * K
  is_last = wid == NW - 1
  n = jnp.where(is_last, C2, C1)
  slab = pl.ds(sid * RPT, RPT)

  @pl.when(jnp.logical_not(is_last))
  def _():
    pltpu.async_copy(ei.at[0, pl.ds(ebase, C1 * K)], sidx, isem)
    pltpu.async_copy(ei.at[1, pl.ds(ebase, C1 * K)], didx, isem)

  @pl.when(is_last)
  def _():
    pltpu.async_copy(ei.at[0, pl.ds(ebase, C2 * K)],
                     sidx.at[pl.ds(0, C2 * K)], isem)
    pltpu.async_copy(ei.at[1, pl.ds(ebase, C2 * K)],
                     didx.at[pl.ds(0, C2 * K)], isem)

  pltpu.sync_copy(u.at[slab], uloc)
  pltpu.sync_copy(dis16.at[slab], dloc)

  def prow(r, carry):
    uloc[r] = uloc[r] * dloc[r]
    return carry

  lax.fori_loop(0, RPT, prow, 0)

  pltpu.sync_copy(uloc, ysp.at[slab])

  @pl.when(cid == 0)
  def _():
    pltpu.sync_copy(uloc, acc.at[slab])

  @pl.when(cid != 0)
  def _():
    pltpu.sync_copy(zeros_h.at[slab], acc.at[slab])

  @pl.when(jnp.logical_not(is_last))
  def _():
    pltpu.make_async_copy(ei.at[0, pl.ds(ebase, C1 * K)], sidx, isem).wait()
    pltpu.make_async_copy(ei.at[1, pl.ds(ebase, C1 * K)], didx, isem).wait()

  @pl.when(is_last)
  def _():
    pltpu.make_async_copy(ei.at[0, pl.ds(ebase, C2 * K)],
                          sidx.at[pl.ds(0, C2 * K)], isem).wait()
    pltpu.make_async_copy(ei.at[1, pl.ds(ebase, C2 * K)],
                          didx.at[pl.ds(0, C2 * K)], isem).wait()

  plsc.subcore_barrier()

  pltpu.async_copy(ysp.at[sidx.at[pl.ds(0, K)]], rows.at[0], gsem)
  pltpu.async_copy(ysp.at[sidx.at[pl.ds(K, K)]], rows.at[1], gsem)

  def body(j, carry):
    b = lax.rem(j, NBUF)

    @pl.when(j >= 2)
    def _():
      pltpu.make_async_copy(
          rows.at[lax.rem(j + 2, NBUF)],
          acc.at[didx.at[pl.ds((j - 2) * K, K)]], ssem).wait()

    @pl.when(j + 2 < n)
    def _():
      pltpu.async_copy(
          ysp.at[sidx.at[pl.ds((j + 2) * K, K)]],
          rows.at[lax.rem(j + 2, NBUF)], gsem)

    pltpu.make_async_copy(
        ysp.at[sidx.at[pl.ds(j * K, K)]], rows.at[b], gsem).wait()
    pltpu.async_copy(rows.at[b], acc.at[didx.at[pl.ds(j * K, K)]], ssem,
                     add=True)
    return carry

  lax.fori_loop(0, n, body, 0)
  pltpu.make_async_copy(
      rows.at[0], acc.at[didx.at[pl.ds(0, K)]], ssem).wait()
  pltpu.make_async_copy(
      rows.at[0], acc.at[didx.at[pl.ds(0, K)]], ssem).wait()
  plsc.subcore_barrier()
  pltpu.sync_copy(acc.at[slab], out.at[cid, slab])


@functools.partial(
    pl.kernel,
    out_type=jax.ShapeDtypeStruct((NC, NP, 16), jnp.float32),
    mesh=plsc.VectorSubcoreMesh(**_MESH),
    scratch_types=[
        pltpu.VMEM((C1 * K,), jnp.int32),
        pltpu.VMEM((K, 16), jnp.float32),
        pltpu.VMEM_SHARED((NA, 16), jnp.float32),
        pltpu.SemaphoreType.DMA,
        pltpu.SemaphoreType.DMA,
    ],
    compiler_params=_SC_PARAMS,
)
def _deg_sc(zeros_nh, ones_kh, ei, out, didx, ones_v, acc, ssem, isem):
  cid = lax.axis_index("c")
  sid = lax.axis_index("s")
  wid = sid * NC + cid
  ebase = wid * C1 * K
  is_last = wid == NW - 1
  n = jnp.where(is_last, C2, C1)

  @pl.when(jnp.logical_not(is_last))
  def _():
    pltpu.async_copy(ei.at[1, pl.ds(ebase, C1 * K)], didx, isem)

  @pl.when(is_last)
  def _():
    pltpu.async_copy(ei.at[1, pl.ds(ebase, C2 * K)],
                     didx.at[pl.ds(0, C2 * K)], isem)

  pltpu.sync_copy(ones_kh, ones_v)
  pltpu.sync_copy(zeros_nh.at[pl.ds(sid * RPT, RPT)],
                  acc.at[pl.ds(sid * RPT, RPT)])

  @pl.when(jnp.logical_not(is_last))
  def _():
    pltpu.make_async_copy(ei.at[1, pl.ds(ebase, C1 * K)], didx, isem).wait()

  @pl.when(is_last)
  def _():
    pltpu.make_async_copy(ei.at[1, pl.ds(ebase, C2 * K)],
                          didx.at[pl.ds(0, C2 * K)], isem).wait()

  plsc.subcore_barrier()

  def body(j, carry):
    @pl.when(j >= 4)
    def _():
      pltpu.make_async_copy(
          ones_v, acc.at[didx.at[pl.ds(0, K)]], ssem).wait()

    pltpu.async_copy(ones_v, acc.at[didx.at[pl.ds(j * K, K)]], ssem,
                     add=True)
    return carry

  lax.fori_loop(0, n, body, 0)
  for _ in range(4):
    pltpu.make_async_copy(ones_v, acc.at[didx.at[pl.ds(0, K)]], ssem).wait()
  plsc.subcore_barrier()
  pltpu.sync_copy(acc.at[pl.ds(sid * RPT, RPT)],
                  out.at[cid, pl.ds(sid * RPT, RPT)])


def _tc0_body(x, w1, u_ref):
  u_ref[0:N, :] = jnp.dot(x[...], w1[...],
                          preferred_element_type=jnp.float32)
  u_ref[N:NP, :] = jnp.zeros((NP - N, H1), jnp.float32)


def _tc2_body(zpr, disp, b1r, w2, u2_ref):
  d16 = disp[...]
  d32 = jnp.concatenate(
      [d16[:, 16 * c:16 * (c + 1)] for c in range(4) for _ in range(2)],
      axis=1)
  z4 = zpr[0, :, :] + zpr[1, :, :]
  h4 = jnp.maximum(z4 * d32 + b1r[...], 0.0)
  parts = [
      jnp.dot(h4[:, 32 * c:32 * (c + 1)], w2[...],
              preferred_element_type=jnp.float32)
      for c in range(4)
  ]
  u2_ref[...] = jnp.concatenate(parts, axis=1)


def _tc3_body(zpr, disp, b2r, wout, bout, embp_ref, logitp_ref):
  z8 = zpr[0, :, :] + zpr[1, :, :]
  emb8 = jnp.maximum(z8 * disp[...] + b2r[...], 0.0)
  embp_ref[...] = emb8
  parts = [
      jnp.dot(emb8[:, 16 * c:16 * (c + 1)], wout[...],
              preferred_element_type=jnp.float32)
      for c in range(8)
  ]
  logitp_ref[...] = jnp.concatenate(parts, axis=1) + bout[0, 0]


def kernel(x, edge_index, W1, b1, W2, b2, Wout, bout):
  zeros_16 = jnp.zeros((NP, 16), jnp.float32)
  zeros_32 = jnp.zeros((NP, H1), jnp.float32)
  ones_kh = jnp.ones((K, 16), jnp.float32)

  degp = _deg_sc(zeros_16, ones_kh, edge_index)

  u1 = pl.pallas_call(
      _tc0_body,
      out_shape=jax.ShapeDtypeStruct((NP, H1), jnp.float32),
  )(x, W1)

  zp1, dis16 = _agg32(u1, degp, edge_index, zeros_32)

  b1r = jnp.tile(b1, 4).reshape(1, 128)
  u2p = pl.pallas_call(
      _tc2_body,
      out_shape=jax.ShapeDtypeStruct((NP // 4, 64), jnp.float32),
  )(zp1.reshape(NC, NP // 4, 128), dis16.reshape(NP // 4, 64), b1r, W2)

  zp2 = _agg16(u2p.reshape(NP, H2), dis16, edge_index, zeros_16)

  b2r = jnp.tile(b2, 8).reshape(1, 128)
  embp, logitp = pl.pallas_call(
      _tc3_body,
      out_shape=[
          jax.ShapeDtypeStruct((NP // 8, 128), jnp.float32),
          jax.ShapeDtypeStruct((NP // 8, 8), jnp.float32),
      ],
  )(zp2.reshape(NC, NP // 8, 128), dis16.reshape(NP // 8, 128), b2r, Wout,
    bout.reshape(1, 1))

  return (logitp.reshape(NP)[:N], embp.reshape(NP, H2)[:N])
